# baseline (speedup 1.0000x reference)
from functools import partial

import jax
import jax.numpy as jnp
from jax.experimental import pallas as pl
from jax.experimental.pallas import tpu as pltpu


def _round_up(x, m):
    return ((x + m - 1) // m) * m


_TCHUNK = 4


def _fused_bilstm_kernel(
    x_any,
    xtr_ref,
    wihf_ref,
    bf_ref,
    wihb_ref,
    bb_ref,
    whhf_ref,
    wt_ref,
    btb_ref,
    w1_ref,
    b1_ref,
    w2_ref,
    b2_ref,
    out_ref,
    cbuf,
    sbuf,
    wihs_ref,
    whhs_ref,
    csem,
    ssem,
    *,
    T: int,
    H: int,
    Bt: int,
):
    Tc = _TCHUNK if T % _TCHUNK == 0 else 1
    NC = T // Tc

    def start_chunk(q):
        pltpu.make_async_copy(
            x_any.at[:, pl.ds(q * Tc, Tc), :], cbuf.at[q % 3], csem.at[q % 3]
        ).start()

    def wait_chunk(q):
        pltpu.make_async_copy(
            x_any.at[:, pl.ds(0, Tc), :], cbuf.at[q % 3], csem.at[q % 3]
        ).wait()

    def start_slab(t):
        q = t // Tc
        pltpu.make_async_copy(
            cbuf.at[q % 3, :, t % Tc, :], sbuf.at[t % 3], ssem.at[t % 3]
        ).start()

    def wait_slab(t):
        pltpu.make_async_copy(
            cbuf.at[0, :, 0, :], sbuf.at[t % 3], ssem.at[t % 3]
        ).wait()

    for q in range(min(3, NC)):
        start_chunk(q)
    wait_chunk(0)
    start_slab(0)
    if T > 1:
        if Tc == 1:
            wait_chunk(1)
        start_slab(1)

    lane = jax.lax.broadcasted_iota(jnp.int32, (1, 4 * H), 1)
    half_mask = jnp.where((lane >= 2 * H) & (lane < 3 * H), 1.0, 0.5)
    wihs_ref[...] = wihf_ref[...] * half_mask
    whhs_ref[...] = whhf_ref[...] * half_mask
    b = bf_ref[...] * half_mask

    whh = whhs_ref[...]

    Bh = Bt // 2

    def lstm_step(x_half, h, c):
        gates = (
            jnp.dot(x_half, wihs_ref[...], preferred_element_type=jnp.float32)
            + jnp.dot(h, whh, preferred_element_type=jnp.float32)
            + b
        )
        ti = jnp.tanh(gates[:, 0:H])
        tf = jnp.tanh(gates[:, H:2 * H])
        g = jnp.tanh(gates[:, 2 * H:3 * H])
        to = jnp.tanh(gates[:, 3 * H:4 * H])
        c = 0.5 * ((1.0 + tf) * c + (1.0 + ti) * g)
        h = (0.5 * (1.0 + to)) * jnp.tanh(c)
        return h, c

    h0 = jnp.zeros((Bh, H), jnp.float32)
    c0 = jnp.zeros((Bh, H), jnp.float32)
    h1 = jnp.zeros((Bh, H), jnp.float32)
    c1 = jnp.zeros((Bh, H), jnp.float32)
    x_t = None
    for t in range(T):
        wait_slab(t)
        x_t = sbuf[t % 3]
        nxt = t + 2
        if nxt < T:
            if nxt % Tc == 0:
                wait_chunk(nxt // Tc)
            start_slab(nxt)
        if t % Tc == Tc - 1 and t // Tc + 3 < NC:
            start_chunk(t // Tc + 3)
        h0, c0 = lstm_step(x_t[0:Bh], h0, c0)
        h1, c1 = lstm_step(x_t[Bh:Bt], h1, c1)
    h = jnp.concatenate([h0, h1], axis=0)

    gb = (
        jnp.dot(x_t, wihb_ref[...], preferred_element_type=jnp.float32)
        + bb_ref[...]
    )
    ti_b = jnp.tanh(gb[:, 0:H] * 0.5)
    g_b = jnp.tanh(gb[:, 2 * H:3 * H])
    to_b = jnp.tanh(gb[:, 3 * H:4 * H] * 0.5)
    c_b = (0.5 * (1.0 + ti_b)) * g_b
    h_b = (0.5 * (1.0 + to_b)) * jnp.tanh(c_b)

    track = jnp.maximum(
        jnp.dot(xtr_ref[...], wt_ref[...], preferred_element_type=jnp.float32)
        + btb_ref[...],
        0.0,
    )

    pre = (
        jnp.dot(h, w1_ref[0:H, :], preferred_element_type=jnp.float32)
        + jnp.dot(h_b, w1_ref[H:2 * H, :], preferred_element_type=jnp.float32)
        + jnp.dot(track, w1_ref[2 * H:3 * H, :], preferred_element_type=jnp.float32)
        + b1_ref[...]
    )
    hidden = jnp.maximum(pre, 0.0)
    out = (
        jnp.dot(hidden, w2_ref[...], preferred_element_type=jnp.float32)
        + b2_ref[...]
    )
    out_ref[...] = out.astype(out_ref.dtype)


@jax.jit
def kernel(x_seq, x_track, wih_f, b_f, wih_b, b_b, whh_f, wt, bt, w1, b1, w2p, b2p):
    B, T, Din = x_seq.shape
    Dtrk = x_track.shape[1]
    H = whh_f.shape[0]

    B_pad = _round_up(B, 8)
    if B_pad != B:
        x_seq = jnp.pad(x_seq, ((0, B_pad - B), (0, 0), (0, 0)))
        x_track = jnp.pad(x_track, ((0, B_pad - B), (0, 0)))

    Tc = _TCHUNK if T % _TCHUNK == 0 else 1

    out = pl.pallas_call(
        partial(_fused_bilstm_kernel, T=T, H=H, Bt=B_pad),
        out_shape=jax.ShapeDtypeStruct((B_pad, 128), jnp.float32),
        grid=(1,),
        in_specs=[
            pl.BlockSpec(memory_space=pltpu.MemorySpace.HBM),
            pl.BlockSpec((B_pad, Dtrk), lambda i: (0, 0)),
            pl.BlockSpec((Din, 4 * H), lambda i: (0, 0)),
            pl.BlockSpec((1, 4 * H), lambda i: (0, 0)),
            pl.BlockSpec((Din, 4 * H), lambda i: (0, 0)),
            pl.BlockSpec((1, 4 * H), lambda i: (0, 0)),
            pl.BlockSpec((H, 4 * H), lambda i: (0, 0)),
            pl.BlockSpec((Dtrk, H), lambda i: (0, 0)),
            pl.BlockSpec((1, H), lambda i: (0, 0)),
            pl.BlockSpec((3 * H, 64), lambda i: (0, 0)),
            pl.BlockSpec((1, 64), lambda i: (0, 0)),
            pl.BlockSpec((64, 128), lambda i: (0, 0)),
            pl.BlockSpec((1, 128), lambda i: (0, 0)),
        ],
        out_specs=pl.BlockSpec((B_pad, 128), lambda i: (0, 0)),
        scratch_shapes=[
            pltpu.VMEM((3, B_pad, Tc, Din), jnp.float32),
            pltpu.VMEM((3, B_pad, Din), jnp.float32),
            pltpu.VMEM((Din, 4 * H), jnp.float32),
            pltpu.VMEM((H, 4 * H), jnp.float32),
            pltpu.SemaphoreType.DMA((3,)),
            pltpu.SemaphoreType.DMA((3,)),
        ],
        compiler_params=pltpu.CompilerParams(
            dimension_semantics=("arbitrary",),
            vmem_limit_bytes=64 * 1024 * 1024,
        ),
    )(x_seq, x_track, wih_f, b_f, wih_b, b_b, whh_f, wt, bt, w1, b1, w2p, b2p)

    return out[:B, :3]


# speedup vs baseline: 1.7829x; 1.7829x over previous
from functools import partial

import jax
import jax.numpy as jnp
from jax.experimental import pallas as pl
from jax.experimental.pallas import tpu as pltpu


def _round_up(x, m):
    return ((x + m - 1) // m) * m


_TCHUNK = 8


def _fused_bilstm_kernel(
    x_any,
    xtr_ref,
    wihf_ref,
    bf_ref,
    wihb_ref,
    bb_ref,
    whhf_ref,
    wt_ref,
    btb_ref,
    w1_ref,
    b1_ref,
    w2_ref,
    b2_ref,
    out_ref,
    cbuf,
    wihs_ref,
    whhs_ref,
    csem,
    *,
    T: int,
    H: int,
    Bt: int,
):
    Tc = _TCHUNK if T % _TCHUNK == 0 else 1
    NC = T // Tc

    def start_chunk(q):
        pltpu.make_async_copy(
            x_any.at[:, pl.ds(q * Tc, Tc), :], cbuf.at[q % 3], csem.at[q % 3]
        ).start()

    def wait_chunk(q):
        pltpu.make_async_copy(
            x_any.at[:, pl.ds(0, Tc), :], cbuf.at[q % 3], csem.at[q % 3]
        ).wait()

    for q in range(min(3, NC)):
        start_chunk(q)

    lane = jax.lax.broadcasted_iota(jnp.int32, (1, 4 * H), 1)
    half_mask = jnp.where((lane >= 2 * H) & (lane < 3 * H), 1.0, 0.5)
    wihs_ref[...] = wihf_ref[...] * half_mask
    whhs_ref[...] = whhf_ref[...] * half_mask
    b = bf_ref[...] * half_mask

    whh = whhs_ref[...]

    Bh = Bt // 2

    def lstm_step(x_half, h, c):
        gates = (
            jnp.dot(x_half, wihs_ref[...], preferred_element_type=jnp.float32)
            + jnp.dot(h, whh, preferred_element_type=jnp.float32)
            + b
        )
        ti = jnp.tanh(gates[:, 0:H])
        tf = jnp.tanh(gates[:, H:2 * H])
        g = jnp.tanh(gates[:, 2 * H:3 * H])
        to = jnp.tanh(gates[:, 3 * H:4 * H])
        c = 0.5 * ((1.0 + tf) * c + (1.0 + ti) * g)
        h = (0.5 * (1.0 + to)) * jnp.tanh(c)
        return h, c

    h0 = jnp.zeros((Bh, H), jnp.float32)
    c0 = jnp.zeros((Bh, H), jnp.float32)
    h1 = jnp.zeros((Bh, H), jnp.float32)
    c1 = jnp.zeros((Bh, H), jnp.float32)
    xa = xb = None
    for t in range(T):
        q, r = divmod(t, Tc)
        if r == 0:
            wait_chunk(q)
        xa = cbuf[q % 3, 0:Bh, r, :]
        xb = cbuf[q % 3, Bh:Bt, r, :]
        h0, c0 = lstm_step(xa, h0, c0)
        h1, c1 = lstm_step(xb, h1, c1)
        if r == Tc - 1 and q + 3 < NC:
            start_chunk(q + 3)

    def head_half(x_last, h, rows):
        gb = (
            jnp.dot(x_last, wihb_ref[...], preferred_element_type=jnp.float32)
            + bb_ref[...]
        )
        ti_b = jnp.tanh(gb[:, 0:H] * 0.5)
        g_b = jnp.tanh(gb[:, 2 * H:3 * H])
        to_b = jnp.tanh(gb[:, 3 * H:4 * H] * 0.5)
        c_b = (0.5 * (1.0 + ti_b)) * g_b
        h_b = (0.5 * (1.0 + to_b)) * jnp.tanh(c_b)

        track = jnp.maximum(
            jnp.dot(xtr_ref[rows, :], wt_ref[...], preferred_element_type=jnp.float32)
            + btb_ref[...],
            0.0,
        )

        pre = (
            jnp.dot(h, w1_ref[0:H, :], preferred_element_type=jnp.float32)
            + jnp.dot(h_b, w1_ref[H:2 * H, :], preferred_element_type=jnp.float32)
            + jnp.dot(track, w1_ref[2 * H:3 * H, :], preferred_element_type=jnp.float32)
            + b1_ref[...]
        )
        hidden = jnp.maximum(pre, 0.0)
        out = (
            jnp.dot(hidden, w2_ref[...], preferred_element_type=jnp.float32)
            + b2_ref[...]
        )
        out_ref[rows, :] = out.astype(out_ref.dtype)

    head_half(xa, h0, pl.ds(0, Bh))
    head_half(xb, h1, pl.ds(Bh, Bh))


@jax.jit
def kernel(x_seq, x_track, wih_f, b_f, wih_b, b_b, whh_f, wt, bt, w1, b1, w2p, b2p):
    B, T, Din = x_seq.shape
    Dtrk = x_track.shape[1]
    H = whh_f.shape[0]

    B_pad = _round_up(B, 8)
    if B_pad != B:
        x_seq = jnp.pad(x_seq, ((0, B_pad - B), (0, 0), (0, 0)))
        x_track = jnp.pad(x_track, ((0, B_pad - B), (0, 0)))

    Tc = _TCHUNK if T % _TCHUNK == 0 else 1

    out = pl.pallas_call(
        partial(_fused_bilstm_kernel, T=T, H=H, Bt=B_pad),
        out_shape=jax.ShapeDtypeStruct((B_pad, 128), jnp.float32),
        grid=(1,),
        in_specs=[
            pl.BlockSpec(memory_space=pltpu.MemorySpace.HBM),
            pl.BlockSpec((B_pad, Dtrk), lambda i: (0, 0)),
            pl.BlockSpec((Din, 4 * H), lambda i: (0, 0)),
            pl.BlockSpec((1, 4 * H), lambda i: (0, 0)),
            pl.BlockSpec((Din, 4 * H), lambda i: (0, 0)),
            pl.BlockSpec((1, 4 * H), lambda i: (0, 0)),
            pl.BlockSpec((H, 4 * H), lambda i: (0, 0)),
            pl.BlockSpec((Dtrk, H), lambda i: (0, 0)),
            pl.BlockSpec((1, H), lambda i: (0, 0)),
            pl.BlockSpec((3 * H, 64), lambda i: (0, 0)),
            pl.BlockSpec((1, 64), lambda i: (0, 0)),
            pl.BlockSpec((64, 128), lambda i: (0, 0)),
            pl.BlockSpec((1, 128), lambda i: (0, 0)),
        ],
        out_specs=pl.BlockSpec((B_pad, 128), lambda i: (0, 0)),
        scratch_shapes=[
            pltpu.VMEM((3, B_pad, Tc, Din), jnp.float32),
            pltpu.VMEM((Din, 4 * H), jnp.float32),
            pltpu.VMEM((H, 4 * H), jnp.float32),
            pltpu.SemaphoreType.DMA((3,)),
        ],
        compiler_params=pltpu.CompilerParams(
            dimension_semantics=("arbitrary",),
            vmem_limit_bytes=64 * 1024 * 1024,
        ),
    )(x_seq, x_track, wih_f, b_f, wih_b, b_b, whh_f, wt, bt, w1, b1, w2p, b2p)

    return out[:B, :3]


# speedup vs baseline: 1.7833x; 1.0002x over previous
from functools import partial

import jax
import jax.numpy as jnp
from jax.experimental import pallas as pl
from jax.experimental.pallas import tpu as pltpu


def _round_up(x, m):
    return ((x + m - 1) // m) * m


_TCHUNK = 8


def _fused_bilstm_kernel(
    x_any,
    xtr_ref,
    wihf_ref,
    bf_ref,
    wihb_ref,
    bb_ref,
    whhf_ref,
    wt_ref,
    btb_ref,
    w1_ref,
    b1_ref,
    w2_ref,
    b2_ref,
    out_ref,
    cbuf,
    wihs_ref,
    whhs_ref,
    csem,
    *,
    T: int,
    H: int,
    Bt: int,
):
    Tc = _TCHUNK if T % _TCHUNK == 0 else 1
    NC = T // Tc

    def start_chunk(q):
        pltpu.make_async_copy(
            x_any.at[:, pl.ds(q * Tc, Tc), :], cbuf.at[q % 3], csem.at[q % 3]
        ).start()

    def wait_chunk(q):
        pltpu.make_async_copy(
            x_any.at[:, pl.ds(0, Tc), :], cbuf.at[q % 3], csem.at[q % 3]
        ).wait()

    for q in range(min(3, NC)):
        start_chunk(q)

    lane = jax.lax.broadcasted_iota(jnp.int32, (1, 4 * H), 1)
    half_mask = jnp.where((lane >= 2 * H) & (lane < 3 * H), 1.0, 0.5)
    wihs_ref[...] = wihf_ref[...] * half_mask
    whhs_ref[...] = whhf_ref[...] * half_mask
    b = bf_ref[...] * half_mask

    whh = whhs_ref[...]

    Bh = Bt // 4

    def lstm_step(x_half, h, c):
        gates = (
            jnp.dot(x_half, wihs_ref[...], preferred_element_type=jnp.float32)
            + jnp.dot(h, whh, preferred_element_type=jnp.float32)
            + b
        )
        ti = jnp.tanh(gates[:, 0:H])
        tf = jnp.tanh(gates[:, H:2 * H])
        g = jnp.tanh(gates[:, 2 * H:3 * H])
        to = jnp.tanh(gates[:, 3 * H:4 * H])
        c = 0.5 * ((1.0 + tf) * c + (1.0 + ti) * g)
        h = (0.5 * (1.0 + to)) * jnp.tanh(c)
        return h, c

    hs = [jnp.zeros((Bh, H), jnp.float32) for _ in range(4)]
    cs = [jnp.zeros((Bh, H), jnp.float32) for _ in range(4)]
    xs = [None] * 4
    for t in range(T):
        q, r = divmod(t, Tc)
        if r == 0:
            wait_chunk(q)
        for j in range(4):
            xs[j] = cbuf[q % 3, j * Bh:(j + 1) * Bh, r, :]
            hs[j], cs[j] = lstm_step(xs[j], hs[j], cs[j])
        if r == Tc - 1 and q + 3 < NC:
            start_chunk(q + 3)

    def head_half(x_last, h, rows):
        gb = (
            jnp.dot(x_last, wihb_ref[...], preferred_element_type=jnp.float32)
            + bb_ref[...]
        )
        ti_b = jnp.tanh(gb[:, 0:H] * 0.5)
        g_b = jnp.tanh(gb[:, 2 * H:3 * H])
        to_b = jnp.tanh(gb[:, 3 * H:4 * H] * 0.5)
        c_b = (0.5 * (1.0 + ti_b)) * g_b
        h_b = (0.5 * (1.0 + to_b)) * jnp.tanh(c_b)

        track = jnp.maximum(
            jnp.dot(xtr_ref[rows, :], wt_ref[...], preferred_element_type=jnp.float32)
            + btb_ref[...],
            0.0,
        )

        pre = (
            jnp.dot(h, w1_ref[0:H, :], preferred_element_type=jnp.float32)
            + jnp.dot(h_b, w1_ref[H:2 * H, :], preferred_element_type=jnp.float32)
            + jnp.dot(track, w1_ref[2 * H:3 * H, :], preferred_element_type=jnp.float32)
            + b1_ref[...]
        )
        hidden = jnp.maximum(pre, 0.0)
        out = (
            jnp.dot(hidden, w2_ref[...], preferred_element_type=jnp.float32)
            + b2_ref[...]
        )
        out_ref[rows, :] = out.astype(out_ref.dtype)

    for j in range(4):
        head_half(xs[j], hs[j], pl.ds(j * Bh, Bh))


@jax.jit
def kernel(x_seq, x_track, wih_f, b_f, wih_b, b_b, whh_f, wt, bt, w1, b1, w2p, b2p):
    B, T, Din = x_seq.shape
    Dtrk = x_track.shape[1]
    H = whh_f.shape[0]

    B_pad = _round_up(B, 8)
    if B_pad != B:
        x_seq = jnp.pad(x_seq, ((0, B_pad - B), (0, 0), (0, 0)))
        x_track = jnp.pad(x_track, ((0, B_pad - B), (0, 0)))

    Tc = _TCHUNK if T % _TCHUNK == 0 else 1

    out = pl.pallas_call(
        partial(_fused_bilstm_kernel, T=T, H=H, Bt=B_pad),
        out_shape=jax.ShapeDtypeStruct((B_pad, 128), jnp.float32),
        grid=(1,),
        in_specs=[
            pl.BlockSpec(memory_space=pltpu.MemorySpace.HBM),
            pl.BlockSpec((B_pad, Dtrk), lambda i: (0, 0)),
            pl.BlockSpec((Din, 4 * H), lambda i: (0, 0)),
            pl.BlockSpec((1, 4 * H), lambda i: (0, 0)),
            pl.BlockSpec((Din, 4 * H), lambda i: (0, 0)),
            pl.BlockSpec((1, 4 * H), lambda i: (0, 0)),
            pl.BlockSpec((H, 4 * H), lambda i: (0, 0)),
            pl.BlockSpec((Dtrk, H), lambda i: (0, 0)),
            pl.BlockSpec((1, H), lambda i: (0, 0)),
            pl.BlockSpec((3 * H, 64), lambda i: (0, 0)),
            pl.BlockSpec((1, 64), lambda i: (0, 0)),
            pl.BlockSpec((64, 128), lambda i: (0, 0)),
            pl.BlockSpec((1, 128), lambda i: (0, 0)),
        ],
        out_specs=pl.BlockSpec((B_pad, 128), lambda i: (0, 0)),
        scratch_shapes=[
            pltpu.VMEM((3, B_pad, Tc, Din), jnp.float32),
            pltpu.VMEM((Din, 4 * H), jnp.float32),
            pltpu.VMEM((H, 4 * H), jnp.float32),
            pltpu.SemaphoreType.DMA((3,)),
        ],
        compiler_params=pltpu.CompilerParams(
            dimension_semantics=("arbitrary",),
            vmem_limit_bytes=64 * 1024 * 1024,
        ),
    )(x_seq, x_track, wih_f, b_f, wih_b, b_b, whh_f, wt, bt, w1, b1, w2p, b2p)

    return out[:B, :3]


# speedup vs baseline: 1.7953x; 1.0068x over previous
from functools import partial

import jax
import jax.numpy as jnp
from jax.experimental import pallas as pl
from jax.experimental.pallas import tpu as pltpu


def _round_up(x, m):
    return ((x + m - 1) // m) * m


_TCHUNK = 8


def _fused_bilstm_kernel(
    x_any,
    xtr_ref,
    wihf_ref,
    bf_ref,
    wihb_ref,
    bb_ref,
    whhf_ref,
    wt_ref,
    btb_ref,
    w1_ref,
    b1_ref,
    w2_ref,
    b2_ref,
    out_ref,
    cbuf,
    wcat_ref,
    csem,
    *,
    T: int,
    H: int,
    Bt: int,
):
    Tc = _TCHUNK if T % _TCHUNK == 0 else 1
    NC = T // Tc

    def start_chunk(q):
        pltpu.make_async_copy(
            x_any.at[:, pl.ds(q * Tc, Tc), :], cbuf.at[q % 3], csem.at[q % 3]
        ).start()

    def wait_chunk(q):
        pltpu.make_async_copy(
            x_any.at[:, pl.ds(0, Tc), :], cbuf.at[q % 3], csem.at[q % 3]
        ).wait()

    for q in range(min(3, NC)):
        start_chunk(q)

    Din = x_any.shape[2]
    Kc = _round_up(Din + H + 1, 128)

    lane = jax.lax.broadcasted_iota(jnp.int32, (1, 4 * H), 1)
    half_mask = jnp.where((lane >= 2 * H) & (lane < 3 * H), 1.0, 0.5)
    wcat_ref[0:Din, :] = wihf_ref[...] * half_mask
    wcat_ref[Din:Din + H, :] = whhf_ref[...] * half_mask
    row = jax.lax.broadcasted_iota(jnp.int32, (Kc - Din - H, 4 * H), 0)
    wcat_ref[Din + H:Kc, :] = jnp.where(row == 0, bf_ref[...] * half_mask, 0.0)

    Bh = Bt // 4

    def lstm_step(x_half, h, c):
        ones = jnp.ones((x_half.shape[0], Kc - Din - H), jnp.float32)
        xh = jnp.concatenate([x_half, h, ones], axis=1)
        gates = jnp.dot(xh, wcat_ref[...], preferred_element_type=jnp.float32)
        ti = jnp.tanh(gates[:, 0:H])
        tf = jnp.tanh(gates[:, H:2 * H])
        g = jnp.tanh(gates[:, 2 * H:3 * H])
        to = jnp.tanh(gates[:, 3 * H:4 * H])
        c = 0.5 * ((1.0 + tf) * c + (1.0 + ti) * g)
        h = (0.5 * (1.0 + to)) * jnp.tanh(c)
        return h, c

    hs = [jnp.zeros((Bh, H), jnp.float32) for _ in range(4)]
    cs = [jnp.zeros((Bh, H), jnp.float32) for _ in range(4)]
    xs = [None] * 4
    for t in range(T):
        q, r = divmod(t, Tc)
        if r == 0:
            wait_chunk(q)
        for j in range(4):
            xs[j] = cbuf[q % 3, j * Bh:(j + 1) * Bh, r, :]
            hs[j], cs[j] = lstm_step(xs[j], hs[j], cs[j])
        if r == Tc - 1 and q + 3 < NC:
            start_chunk(q + 3)

    def head_half(x_last, h, rows):
        gb = (
            jnp.dot(x_last, wihb_ref[...], preferred_element_type=jnp.float32)
            + bb_ref[...]
        )
        ti_b = jnp.tanh(gb[:, 0:H] * 0.5)
        g_b = jnp.tanh(gb[:, 2 * H:3 * H])
        to_b = jnp.tanh(gb[:, 3 * H:4 * H] * 0.5)
        c_b = (0.5 * (1.0 + ti_b)) * g_b
        h_b = (0.5 * (1.0 + to_b)) * jnp.tanh(c_b)

        track = jnp.maximum(
            jnp.dot(xtr_ref[rows, :], wt_ref[...], preferred_element_type=jnp.float32)
            + btb_ref[...],
            0.0,
        )

        pre = (
            jnp.dot(h, w1_ref[0:H, :], preferred_element_type=jnp.float32)
            + jnp.dot(h_b, w1_ref[H:2 * H, :], preferred_element_type=jnp.float32)
            + jnp.dot(track, w1_ref[2 * H:3 * H, :], preferred_element_type=jnp.float32)
            + b1_ref[...]
        )
        hidden = jnp.maximum(pre, 0.0)
        out = (
            jnp.dot(hidden, w2_ref[...], preferred_element_type=jnp.float32)
            + b2_ref[...]
        )
        out_ref[rows, :] = out.astype(out_ref.dtype)

    for j in range(4):
        head_half(xs[j], hs[j], pl.ds(j * Bh, Bh))


@jax.jit
def kernel(x_seq, x_track, wih_f, b_f, wih_b, b_b, whh_f, wt, bt, w1, b1, w2p, b2p):
    B, T, Din = x_seq.shape
    Dtrk = x_track.shape[1]
    H = whh_f.shape[0]

    B_pad = _round_up(B, 8)
    if B_pad != B:
        x_seq = jnp.pad(x_seq, ((0, B_pad - B), (0, 0), (0, 0)))
        x_track = jnp.pad(x_track, ((0, B_pad - B), (0, 0)))

    Tc = _TCHUNK if T % _TCHUNK == 0 else 1

    out = pl.pallas_call(
        partial(_fused_bilstm_kernel, T=T, H=H, Bt=B_pad),
        out_shape=jax.ShapeDtypeStruct((B_pad, 128), jnp.float32),
        grid=(1,),
        in_specs=[
            pl.BlockSpec(memory_space=pltpu.MemorySpace.HBM),
            pl.BlockSpec((B_pad, Dtrk), lambda i: (0, 0)),
            pl.BlockSpec((Din, 4 * H), lambda i: (0, 0)),
            pl.BlockSpec((1, 4 * H), lambda i: (0, 0)),
            pl.BlockSpec((Din, 4 * H), lambda i: (0, 0)),
            pl.BlockSpec((1, 4 * H), lambda i: (0, 0)),
            pl.BlockSpec((H, 4 * H), lambda i: (0, 0)),
            pl.BlockSpec((Dtrk, H), lambda i: (0, 0)),
            pl.BlockSpec((1, H), lambda i: (0, 0)),
            pl.BlockSpec((3 * H, 64), lambda i: (0, 0)),
            pl.BlockSpec((1, 64), lambda i: (0, 0)),
            pl.BlockSpec((64, 128), lambda i: (0, 0)),
            pl.BlockSpec((1, 128), lambda i: (0, 0)),
        ],
        out_specs=pl.BlockSpec((B_pad, 128), lambda i: (0, 0)),
        scratch_shapes=[
            pltpu.VMEM((3, B_pad, Tc, Din), jnp.float32),
            pltpu.VMEM((_round_up(Din + H + 1, 128), 4 * H), jnp.float32),
            pltpu.SemaphoreType.DMA((3,)),
        ],
        compiler_params=pltpu.CompilerParams(
            dimension_semantics=("arbitrary",),
            vmem_limit_bytes=64 * 1024 * 1024,
        ),
    )(x_seq, x_track, wih_f, b_f, wih_b, b_b, whh_f, wt, bt, w1, b1, w2p, b2p)

    return out[:B, :3]
